# baseline (speedup 1.0000x reference)
import jax
import jax.numpy as jnp
from jax.experimental import pallas as pl

B = 262144
NUM_TILES = 64
SIG_DIM = 16
RB = 65536


def _body(sigt_ref, tsig_ref, scorest_ref, idx_ref):
    st = sigt_ref[...]
    t = tsig_ref[...]
    sc = jax.lax.dot_general(
        t, st, (((1,), (0,)), ((), ())),
        preferred_element_type=jnp.float32)
    scorest_ref[...] = sc
    mx = jnp.max(sc, axis=0, keepdims=True)
    iota = jax.lax.broadcasted_iota(jnp.int32, sc.shape, 0)
    idx_ref[...] = jnp.min(jnp.where(sc == mx, iota, NUM_TILES), axis=0)


def kernel(sig, tile_signatures):
    sigt = sig.T
    scorest, idx = pl.pallas_call(
        _body,
        grid=(B // RB,),
        in_specs=[
            pl.BlockSpec((SIG_DIM, RB), lambda i: (0, i)),
            pl.BlockSpec((NUM_TILES, SIG_DIM), lambda i: (0, 0)),
        ],
        out_specs=[
            pl.BlockSpec((NUM_TILES, RB), lambda i: (0, i)),
            pl.BlockSpec((RB,), lambda i: (i,)),
        ],
        out_shape=[
            jax.ShapeDtypeStruct((NUM_TILES, B), jnp.float32),
            jax.ShapeDtypeStruct((B,), jnp.int32),
        ],
    )(sigt, tile_signatures)
    return scorest.T, idx


# speedup vs baseline: 1.0218x; 1.0218x over previous
import jax
import jax.numpy as jnp
from jax.experimental import pallas as pl

B = 262144
NUM_TILES = 64
SIG_DIM = 16
RB = 32768


def _body(sigt_ref, tsig_ref, scorest_ref, idx_ref):
    st = sigt_ref[...]
    t = tsig_ref[...]
    sc = jax.lax.dot_general(
        t, st, (((1,), (0,)), ((), ())),
        preferred_element_type=jnp.float32)
    scorest_ref[...] = sc
    mx = jnp.max(sc, axis=0, keepdims=True)
    iota = jax.lax.broadcasted_iota(jnp.int32, sc.shape, 0)
    idx_ref[...] = jnp.min(jnp.where(sc == mx, iota, NUM_TILES), axis=0)


def kernel(sig, tile_signatures):
    sigt = sig.T
    scorest, idx = pl.pallas_call(
        _body,
        grid=(B // RB,),
        in_specs=[
            pl.BlockSpec((SIG_DIM, RB), lambda i: (0, i)),
            pl.BlockSpec((NUM_TILES, SIG_DIM), lambda i: (0, 0)),
        ],
        out_specs=[
            pl.BlockSpec((NUM_TILES, RB), lambda i: (0, i)),
            pl.BlockSpec((RB,), lambda i: (i,)),
        ],
        out_shape=[
            jax.ShapeDtypeStruct((NUM_TILES, B), jnp.float32),
            jax.ShapeDtypeStruct((B,), jnp.int32),
        ],
    )(sigt, tile_signatures)
    return scorest.T, idx
